# baseline (speedup 1.0000x reference)
import functools

import jax
import jax.numpy as jnp
from jax import lax
from jax.experimental import pallas as pl
from jax.experimental.pallas import tpu as pltpu
from jax.experimental.pallas import tpu_sc as plsc

NC = 2
NS = 16
L = 16
NW = NC * NS
K = 128


def _ceil_to(a, m):
    return (a + m - 1) // m * m


def _deg_body(B, RPAD, de_hbm, degp_hbm, didx_v, dl_v, deg_v, db_v, idr_v,
              deg_sp, semi):
    c = lax.axis_index("c")
    s = lax.axis_index("s")
    w = s * NC + c
    R2 = RPAD // 2
    DR = RPAD // 128
    ci = pltpu.async_copy(de_hbm.at[pl.ds(w * B, B)], didx_v, semi)

    def zero_rows(ref, nrow, ncol):
        def zr(i, _):
            for j in range(ncol // L):
                ref[i, pl.ds(j * L, L)] = jnp.zeros((L,), jnp.float32)
            return 0
        lax.fori_loop(0, nrow, zr, 0)

    zero_rows(deg_v, DR, 128)
    for g in range(DR // L):
        idr_v[pl.ds(g * L, L)] = lax.iota(jnp.int32, L) + g * L
    @pl.when(s == 0)
    def _():
        pltpu.sync_copy(deg_v, deg_sp)
    plsc.subcore_barrier()

    lane = lax.iota(jnp.int32, L)
    for p in range(2):
        lo = p * R2

        def zf(i, _):
            for j in range(8):
                dl_v[pl.ds(i * 128 + j * L, L)] = jnp.zeros((L,), jnp.float32)
            return 0

        lax.fori_loop(0, R2 // 8, zf, 0)
        if p == 0:
            ci.wait()

        def grp(i, _):
            v = didx_v[i // (K // L), pl.ds((i % (K // L)) * L, L)]
            idx = lax.shift_right_logical(v, 16)
            m = (idx >= lo) & (idx < lo + R2)
            fi = jnp.where(m, (idx - lo) * L + lane, R2 * L + lane)
            cur = plsc.load_gather(dl_v, [fi])
            plsc.store_scatter(dl_v, [fi], cur + 1.0)
            return 0

        lax.fori_loop(0, B * (K // L), grp, 0)

        def red(rr, _):
            for j in range(8):
                p0 = rr * 128 + j * L
                acc = jnp.zeros((L,), jnp.float32)
                for l in range(L):
                    acc = acc + plsc.load_gather(dl_v, [(p0 + lane) * L + l])
                deg_v[lo // 128 + rr, pl.ds(j * L, L)] = acc
            return 0

        lax.fori_loop(0, R2 // 128, red, 0)

    pltpu.sync_copy(deg_v, deg_sp.at[idr_v], add=True)
    plsc.subcore_barrier()

    @pl.when(s == 0)
    def _():
        pltpu.sync_copy(deg_sp, db_v)
        pltpu.sync_copy(db_v, degp_hbm.at[c])


def _scat_body(B0, B1, RPT, D, g_hbm, ep_hbm, outp_hbm,
               eidx_v, rows0_v, rows1_v, sidx0_v, sidx1_v, didx0_v, didx1_v,
               out_sp, semi, sem0, sem1):
    c = lax.axis_index("c")
    s = lax.axis_index("s")

    def zr(i, _):
        for j in range(D // L):
            rows0_v[i, pl.ds(j * L, L)] = jnp.zeros((L,), jnp.float32)
        return 0

    def unpack(b, sidx, didx):
        for j in range(K // L):
            v = eidx_v[b, pl.ds(j * L, L)]
            sidx[pl.ds(j * L, L)] = v & 0xFFFF
            didx[pl.ds(j * L, L)] = lax.shift_right_logical(v, 16)

    def fire(b, sidx, didx, rows, sem):
        unpack(b, sidx, didx)
        pltpu.async_copy(g_hbm.at[sidx], rows, sem)

    def wait(rows, sem):
        pltpu.make_async_copy(g_hbm.at[sidx0_v], rows, sem).wait()

    def scat(didx, rows):
        pltpu.sync_copy(rows, out_sp.at[didx], add=True)

    def run(base_blk, nb):
        ci = pltpu.async_copy(ep_hbm.at[pl.ds(base_blk, nb)],
                              eidx_v.at[pl.ds(0, nb)], semi)
        lax.fori_loop(0, K, zr, 0)
        for t in range(RPT // K):
            pltpu.sync_copy(rows0_v, out_sp.at[pl.ds(s * RPT + t * K, K)])
        ci.wait()
        plsc.subcore_barrier()

        fire(0, sidx0_v, didx0_v, rows0_v, sem0)
        nfull = (nb - 1) // 2

        def body(sb, _):
            fire(2 * sb + 1, sidx1_v, didx1_v, rows1_v, sem1)
            wait(rows0_v, sem0)
            scat(didx0_v, rows0_v)
            fire(2 * sb + 2, sidx0_v, didx0_v, rows0_v, sem0)
            wait(rows1_v, sem1)
            scat(didx1_v, rows1_v)
            return 0

        lax.fori_loop(0, nfull, body, 0)
        if nb % 2 == 1:
            wait(rows0_v, sem0)
            scat(didx0_v, rows0_v)
        else:
            fire(nb - 1, sidx1_v, didx1_v, rows1_v, sem1)
            wait(rows0_v, sem0)
            scat(didx0_v, rows0_v)
            wait(rows1_v, sem1)
            scat(didx1_v, rows1_v)

    @pl.when(c == 0)
    def _():
        run(s * B0, B0)

    @pl.when(c == 1)
    def _():
        run(NS * B0 + s * B1, B1)

    plsc.subcore_barrier()
    for t in range(RPT // K):
        r0 = s * RPT + t * K
        pltpu.sync_copy(out_sp.at[pl.ds(r0, K)], rows0_v)
        pltpu.sync_copy(rows0_v, outp_hbm.at[c, pl.ds(r0, K)])


def _dinv_col(degp_ref, nrow):
    i = pl.program_id(0)
    nr = nrow // 128
    d = (degp_ref[0, pl.ds(nr * i, nr), :]
         + degp_ref[1, pl.ds(nr * i, nr), :] + 1.0)
    dinv = lax.rsqrt(d)
    ri = lax.broadcasted_iota(jnp.int32, (nrow, 128), 0)
    li = lax.broadcasted_iota(jnp.int32, (nrow, 128), 1)
    hi = ri // 128
    o = jnp.broadcast_to(dinv[0:1, :], (nrow, 128))
    for k in range(1, nr):
        o = jnp.where(hi == k, jnp.broadcast_to(dinv[k:k + 1, :],
                                                (nrow, 128)), o)
    sel = li == (ri % 128)
    return jnp.sum(jnp.where(sel, o, 0.0), axis=1, keepdims=True)


def _scale_body(degp, x, W, g_out, dinvb_out):
    nrow = x.shape[0]
    dinv = _dinv_col(degp, nrow)
    h = jnp.dot(x[...], W[...], preferred_element_type=jnp.float32)
    g_out[...] = h * dinv
    dinvb_out[...] = jnp.broadcast_to(dinv, (nrow, 128))


def _fuse_body(dinvb, p, g, b, W, g2_out):
    dinv = dinvb[...]
    y = dinv * (p[0] + p[1] + g[...]) + b[...]
    a = jnp.maximum(y, 0.0)
    g2_out[...] = dinv * jnp.dot(a, W[...], preferred_element_type=jnp.float32)


def _finish_body(dinvb, q, g2, b, x, out):
    dinv = dinvb[...]
    y = dinv * (q[0] + q[1] + g2[...]) + b[...]
    out[...] = jnp.maximum(y, 0.0) + x[...]


@jax.jit
def kernel(x, edge_index, W1, b1, W2, b2):
    if x.ndim == 3:
        x = jnp.squeeze(x, axis=1)
    N, D = x.shape
    E = edge_index.shape[1]

    RPAD = _ceil_to(N, NS * K)
    RPT = RPAD // NS
    DR = RPAD // 128
    TB = _ceil_to(-(-E // K), 16 * NS)
    BPC = TB // NS
    B0 = max(8, min(BPC - 8, 8 * round(BPC * 0.29 / 8)))
    B1 = BPC - B0
    BD = TB // NW
    E_pad = TB * K
    pad_row = jnp.int32(RPAD - 1)

    se = jnp.concatenate(
        [edge_index[0], jnp.full((E_pad - E,), pad_row, jnp.int32)])
    de = jnp.concatenate(
        [edge_index[1], jnp.full((E_pad - E,), pad_row, jnp.int32)])
    ep = jnp.bitwise_or(jnp.left_shift(de, 16), se).reshape(TB, K)
    xp = jnp.pad(x, ((0, RPAD - N), (0, 0)))

    mesh = plsc.VectorSubcoreMesh(core_axis_name="c", subcore_axis_name="s")

    deg_call = pl.kernel(
        functools.partial(_deg_body, BD, RPAD),
        out_type=jax.ShapeDtypeStruct((NC, DR, 128), jnp.float32),
        mesh=mesh,
        compiler_params=pltpu.CompilerParams(needs_layout_passes=False),
        scratch_types=[
            pltpu.VMEM((BD, K), jnp.int32),
            pltpu.VMEM((RPAD // 2 * L + L,), jnp.float32),
            pltpu.VMEM((DR, 128), jnp.float32),
            pltpu.VMEM((DR, 128), jnp.float32),
            pltpu.VMEM((DR,), jnp.int32),
            pltpu.VMEM_SHARED((DR, 128), jnp.float32),
            pltpu.SemaphoreType.DMA,
        ],
    )
    degp = deg_call(ep)

    scat_call = pl.kernel(
        functools.partial(_scat_body, B0, B1, RPT, D),
        out_type=jax.ShapeDtypeStruct((NC, RPAD, D), jnp.float32),
        mesh=mesh,
        scratch_types=[
            pltpu.VMEM((max(B0, B1), K), jnp.int32),
            pltpu.VMEM((K, D), jnp.float32),
            pltpu.VMEM((K, D), jnp.float32),
            pltpu.VMEM((K,), jnp.int32),
            pltpu.VMEM((K,), jnp.int32),
            pltpu.VMEM((K,), jnp.int32),
            pltpu.VMEM((K,), jnp.int32),
            pltpu.VMEM_SHARED((RPAD, D), jnp.float32),
            pltpu.SemaphoreType.DMA,
            pltpu.SemaphoreType.DMA,
            pltpu.SemaphoreType.DMA,
        ],
    )

    BR = 256
    grid = (RPAD // BR,)
    degp_spec = pl.BlockSpec((NC, DR, 128), lambda i: (0, 0, 0))
    row_spec = pl.BlockSpec((BR, D), lambda i: (i, 0))
    p_spec = pl.BlockSpec((NC, BR, D), lambda i: (0, i, 0))
    w_spec = pl.BlockSpec((D, D), lambda i: (0, 0))
    b_spec = pl.BlockSpec((1, D), lambda i: (0, 0))
    rows_out = jax.ShapeDtypeStruct((RPAD, D), jnp.float32)

    g1, dinvb = pl.pallas_call(
        _scale_body, grid=grid,
        in_specs=[degp_spec, row_spec, w_spec],
        out_specs=[row_spec, row_spec], out_shape=[rows_out, rows_out],
    )(degp, xp, W1)

    p = scat_call(g1, ep)

    g2 = pl.pallas_call(
        _fuse_body, grid=grid,
        in_specs=[row_spec, p_spec, row_spec, b_spec, w_spec],
        out_specs=row_spec, out_shape=rows_out,
    )(dinvb, p, g1, b1.reshape(1, D), W2)

    q = scat_call(g2, ep)

    out = pl.pallas_call(
        _finish_body, grid=grid,
        in_specs=[row_spec, p_spec, row_spec, b_spec, row_spec],
        out_specs=row_spec, out_shape=rows_out,
    )(dinvb, q, g2, b2.reshape(1, D), xp)

    return out[:N]


# speedup vs baseline: 2.8520x; 2.8520x over previous
import functools

import jax
import jax.numpy as jnp
from jax import lax
from jax.experimental import pallas as pl
from jax.experimental.pallas import tpu as pltpu
from jax.experimental.pallas import tpu_sc as plsc

NC = 2
NS = 16
L = 16
NW = NC * NS
K = 128


def _ceil_to(a, m):
    return (a + m - 1) // m * m


def _deg_body(B, RPAD, de_hbm, degp_hbm, didx_v, dl_v, deg_v, db_v, idr_v,
              deg_sp, semi):
    c = lax.axis_index("c")
    s = lax.axis_index("s")
    w = s * NC + c
    R2 = RPAD // 2
    DR = RPAD // 128
    ci = pltpu.async_copy(de_hbm.at[pl.ds(w * B, B)], didx_v, semi)

    def zero_rows(ref, nrow, ncol):
        def zr(i, _):
            for j in range(ncol // L):
                ref[i, pl.ds(j * L, L)] = jnp.zeros((L,), jnp.float32)
            return 0
        lax.fori_loop(0, nrow, zr, 0)

    zero_rows(deg_v, DR, 128)
    for g in range(DR // L):
        idr_v[pl.ds(g * L, L)] = lax.iota(jnp.int32, L) + g * L
    @pl.when(s == 0)
    def _():
        pltpu.sync_copy(deg_v, deg_sp)
    plsc.subcore_barrier()

    lane = lax.iota(jnp.int32, L)
    for p in range(2):
        lo = p * R2

        def zf(i, _):
            for j in range(8):
                dl_v[pl.ds(i * 128 + j * L, L)] = jnp.zeros((L,), jnp.float32)
            return 0

        lax.fori_loop(0, R2 // 8, zf, 0)
        if p == 0:
            ci.wait()

        def grp(i, _):
            v = didx_v[i // (K // L), pl.ds((i % (K // L)) * L, L)]
            idx = lax.shift_right_logical(v, 16)
            m = (idx >= lo) & (idx < lo + R2)
            fi = jnp.where(m, (idx - lo) * L + lane, R2 * L + lane)
            cur = plsc.load_gather(dl_v, [fi])
            plsc.store_scatter(dl_v, [fi], cur + 1.0)
            return 0

        lax.fori_loop(0, B * (K // L), grp, 0)

        def red(rr, _):
            for j in range(8):
                p0 = rr * 128 + j * L
                acc = jnp.zeros((L,), jnp.float32)
                for l in range(L):
                    acc = acc + plsc.load_gather(dl_v, [(p0 + lane) * L + l])
                deg_v[lo // 128 + rr, pl.ds(j * L, L)] = acc
            return 0

        lax.fori_loop(0, R2 // 128, red, 0)

    pltpu.sync_copy(deg_v, deg_sp.at[idr_v], add=True)
    plsc.subcore_barrier()

    @pl.when(s == 0)
    def _():
        pltpu.sync_copy(deg_sp, db_v)
        pltpu.sync_copy(db_v, degp_hbm.at[c])


def _scat_body(B0, B1, RPT, D, g_hbm, ep_hbm, outp_hbm,
               eidx_v, rows0_v, rows1_v, sidx0_v, sidx1_v, didx0_v, didx1_v,
               out_sp, semi, sem0, sem1):
    c = lax.axis_index("c")
    s = lax.axis_index("s")

    def zr(i, _):
        for j in range(D // L):
            rows0_v[i, pl.ds(j * L, L)] = jnp.zeros((L,), jnp.float32)
        return 0

    def unpack(b, sidx, didx):
        for j in range(K // L):
            v = eidx_v[b, pl.ds(j * L, L)]
            sidx[pl.ds(j * L, L)] = v & 0xFFFF
            didx[pl.ds(j * L, L)] = lax.shift_right_logical(v, 16)

    def fire(b, sidx, didx, rows, sem):
        unpack(b, sidx, didx)
        pltpu.async_copy(g_hbm.at[sidx], rows, sem)

    def wait(rows, sem):
        pltpu.make_async_copy(g_hbm.at[sidx0_v], rows, sem).wait()

    def scat(didx, rows):
        pltpu.sync_copy(rows, out_sp.at[didx], add=True)

    def run(base_blk, nb):
        ci = pltpu.async_copy(ep_hbm.at[pl.ds(base_blk, nb)],
                              eidx_v.at[pl.ds(0, nb)], semi)
        lax.fori_loop(0, K, zr, 0)
        for t in range(RPT // K):
            pltpu.sync_copy(rows0_v, out_sp.at[pl.ds(s * RPT + t * K, K)])
        ci.wait()
        plsc.subcore_barrier()

        fire(0, sidx0_v, didx0_v, rows0_v, sem0)
        nfull = (nb - 1) // 2

        def body(sb, _):
            fire(2 * sb + 1, sidx1_v, didx1_v, rows1_v, sem1)
            wait(rows0_v, sem0)
            scat(didx0_v, rows0_v)
            fire(2 * sb + 2, sidx0_v, didx0_v, rows0_v, sem0)
            wait(rows1_v, sem1)
            scat(didx1_v, rows1_v)
            return 0

        lax.fori_loop(0, nfull, body, 0)
        if nb % 2 == 1:
            wait(rows0_v, sem0)
            scat(didx0_v, rows0_v)
        else:
            fire(nb - 1, sidx1_v, didx1_v, rows1_v, sem1)
            wait(rows0_v, sem0)
            scat(didx0_v, rows0_v)
            wait(rows1_v, sem1)
            scat(didx1_v, rows1_v)

    @pl.when(c == 0)
    def _():
        run(s * B0, B0)

    @pl.when(c == 1)
    def _():
        run(NS * B0 + s * B1, B1)

    plsc.subcore_barrier()
    for t in range(RPT // K):
        r0 = s * RPT + t * K
        pltpu.sync_copy(out_sp.at[pl.ds(r0, K)], rows0_v)
        pltpu.sync_copy(rows0_v, outp_hbm.at[c, pl.ds(r0, K)])


def _dinv_col(degp_ref, nrow):
    i = pl.program_id(0)
    nr = nrow // 128
    d = (degp_ref[0, pl.ds(nr * i, nr), :]
         + degp_ref[1, pl.ds(nr * i, nr), :] + 1.0)
    dinv = lax.rsqrt(d)
    ri = lax.broadcasted_iota(jnp.int32, (nrow, 128), 0)
    li = lax.broadcasted_iota(jnp.int32, (nrow, 128), 1)
    hi = ri // 128
    o = jnp.broadcast_to(dinv[0:1, :], (nrow, 128))
    for k in range(1, nr):
        o = jnp.where(hi == k, jnp.broadcast_to(dinv[k:k + 1, :],
                                                (nrow, 128)), o)
    sel = li == (ri % 128)
    return jnp.sum(jnp.where(sel, o, 0.0), axis=1, keepdims=True)


def _scale_body(degp, x, W, g_out, dinvb_out):
    nrow = x.shape[0]
    dinv = _dinv_col(degp, nrow)
    h = jnp.dot(x[...], W[...], preferred_element_type=jnp.float32)
    g_out[...] = h * dinv
    dinvb_out[...] = jnp.broadcast_to(dinv, (nrow, 128))


def _fuse_body(dinvb, p, g, b, W, g2_out):
    dinv = dinvb[...]
    y = dinv * (p[0] + p[1] + g[...]) + b[...]
    a = jnp.maximum(y, 0.0)
    g2_out[...] = dinv * jnp.dot(a, W[...], preferred_element_type=jnp.float32)


def _finish_body(dinvb, q, g2, b, x, out):
    dinv = dinvb[...]
    y = dinv * (q[0] + q[1] + g2[...]) + b[...]
    out[...] = jnp.maximum(y, 0.0) + x[...]


@jax.jit
def kernel(x, edge_index, W1, b1, W2, b2):
    if x.ndim == 3:
        x = jnp.squeeze(x, axis=1)
    N, D = x.shape
    E = edge_index.shape[1]

    RPAD = _ceil_to(N, NS * K)
    RPT = RPAD // NS
    DR = RPAD // 128
    TB = _ceil_to(-(-E // K), 16 * NS)
    BPC = TB // NS
    B0 = BPC // 2
    B1 = BPC - B0
    BD = TB // NW
    E_pad = TB * K
    pad_row = jnp.int32(RPAD - 1)

    pad_i = jnp.arange(E_pad - E, dtype=jnp.int32)
    pad_spread = N + pad_i % jnp.int32(max(RPAD - N, 1))
    se = jnp.concatenate([edge_index[0], pad_spread])
    de = jnp.concatenate([edge_index[1], pad_spread])
    ep = jnp.bitwise_or(jnp.left_shift(de, 16), se).reshape(TB, K)
    xp = jnp.pad(x, ((0, RPAD - N), (0, 0)))

    mesh = plsc.VectorSubcoreMesh(core_axis_name="c", subcore_axis_name="s")

    deg_call = pl.kernel(
        functools.partial(_deg_body, BD, RPAD),
        out_type=jax.ShapeDtypeStruct((NC, DR, 128), jnp.float32),
        mesh=mesh,
        compiler_params=pltpu.CompilerParams(needs_layout_passes=False),
        scratch_types=[
            pltpu.VMEM((BD, K), jnp.int32),
            pltpu.VMEM((RPAD // 2 * L + L,), jnp.float32),
            pltpu.VMEM((DR, 128), jnp.float32),
            pltpu.VMEM((DR, 128), jnp.float32),
            pltpu.VMEM((DR,), jnp.int32),
            pltpu.VMEM_SHARED((DR, 128), jnp.float32),
            pltpu.SemaphoreType.DMA,
        ],
    )
    degp = deg_call(ep)

    scat_call = pl.kernel(
        functools.partial(_scat_body, B0, B1, RPT, D),
        out_type=jax.ShapeDtypeStruct((NC, RPAD, D), jnp.float32),
        mesh=mesh,
        scratch_types=[
            pltpu.VMEM((max(B0, B1), K), jnp.int32),
            pltpu.VMEM((K, D), jnp.float32),
            pltpu.VMEM((K, D), jnp.float32),
            pltpu.VMEM((K,), jnp.int32),
            pltpu.VMEM((K,), jnp.int32),
            pltpu.VMEM((K,), jnp.int32),
            pltpu.VMEM((K,), jnp.int32),
            pltpu.VMEM_SHARED((RPAD, D), jnp.float32),
            pltpu.SemaphoreType.DMA,
            pltpu.SemaphoreType.DMA,
            pltpu.SemaphoreType.DMA,
        ],
    )

    BR = 256
    grid = (RPAD // BR,)
    degp_spec = pl.BlockSpec((NC, DR, 128), lambda i: (0, 0, 0))
    row_spec = pl.BlockSpec((BR, D), lambda i: (i, 0))
    p_spec = pl.BlockSpec((NC, BR, D), lambda i: (0, i, 0))
    w_spec = pl.BlockSpec((D, D), lambda i: (0, 0))
    b_spec = pl.BlockSpec((1, D), lambda i: (0, 0))
    rows_out = jax.ShapeDtypeStruct((RPAD, D), jnp.float32)

    g1, dinvb = pl.pallas_call(
        _scale_body, grid=grid,
        in_specs=[degp_spec, row_spec, w_spec],
        out_specs=[row_spec, row_spec], out_shape=[rows_out, rows_out],
    )(degp, xp, W1)

    p = scat_call(g1, ep)

    g2 = pl.pallas_call(
        _fuse_body, grid=grid,
        in_specs=[row_spec, p_spec, row_spec, b_spec, w_spec],
        out_specs=row_spec, out_shape=rows_out,
    )(dinvb, p, g1, b1.reshape(1, D), W2)

    q = scat_call(g2, ep)

    out = pl.pallas_call(
        _finish_body, grid=grid,
        in_specs=[row_spec, p_spec, row_spec, b_spec, row_spec],
        out_specs=row_spec, out_shape=rows_out,
    )(dinvb, q, g2, b2.reshape(1, D), xp)

    return out[:N]


# speedup vs baseline: 3.1013x; 1.0874x over previous
import functools

import jax
import jax.numpy as jnp
from jax import lax
from jax.experimental import pallas as pl
from jax.experimental.pallas import tpu as pltpu
from jax.experimental.pallas import tpu_sc as plsc

NC = 2
NS = 16
L = 16
NW = NC * NS
K = 128


def _ceil_to(a, m):
    return (a + m - 1) // m * m


def _deg_body(B, RPAD, de_hbm, degp_hbm, didx_v, dl_v, deg_v, db_v, idr_v,
              deg_sp, semi):
    c = lax.axis_index("c")
    s = lax.axis_index("s")
    w = s * NC + c
    R2 = RPAD // 2
    DR = RPAD // 128
    ci = pltpu.async_copy(de_hbm.at[pl.ds(w * B, B)], didx_v, semi)

    def zero_rows(ref, nrow, ncol):
        def zr(i, _):
            for j in range(ncol // L):
                ref[i, pl.ds(j * L, L)] = jnp.zeros((L,), jnp.float32)
            return 0
        lax.fori_loop(0, nrow, zr, 0)

    zero_rows(deg_v, DR, 128)
    for g in range(DR // L):
        idr_v[pl.ds(g * L, L)] = lax.iota(jnp.int32, L) + g * L
    @pl.when(s == 0)
    def _():
        pltpu.sync_copy(deg_v, deg_sp)
    plsc.subcore_barrier()

    lane = lax.iota(jnp.int32, L)
    for p in range(2):
        lo = p * R2

        def zf(i, _):
            for j in range(8):
                dl_v[pl.ds(i * 128 + j * L, L)] = jnp.zeros((L,), jnp.float32)
            return 0

        lax.fori_loop(0, R2 // 8, zf, 0)
        if p == 0:
            ci.wait()

        def grp(i, _):
            v = didx_v[i // (K // L), pl.ds((i % (K // L)) * L, L)]
            idx = lax.shift_right_logical(v, 16)
            m = (idx >= lo) & (idx < lo + R2)
            fi = jnp.where(m, (idx - lo) * L + lane, R2 * L + lane)
            cur = plsc.load_gather(dl_v, [fi])
            plsc.store_scatter(dl_v, [fi], cur + 1.0)
            return 0

        lax.fori_loop(0, B * (K // L), grp, 0)

        def red(rr, _):
            for j in range(8):
                p0 = rr * 128 + j * L
                acc = jnp.zeros((L,), jnp.float32)
                for l in range(L):
                    acc = acc + plsc.load_gather(dl_v, [(p0 + lane) * L + l])
                deg_v[lo // 128 + rr, pl.ds(j * L, L)] = acc
            return 0

        lax.fori_loop(0, R2 // 128, red, 0)

    pltpu.sync_copy(deg_v, deg_sp.at[idr_v], add=True)
    plsc.subcore_barrier()

    @pl.when(s == 0)
    def _():
        pltpu.sync_copy(deg_sp, db_v)
        pltpu.sync_copy(db_v, degp_hbm.at[c])


def _scat_body(B0, B1, RPT, D, g_hbm, ep_hbm, outp_hbm,
               eidx_v, rows0_v, rows1_v, sidx0_v, sidx1_v, didx0_v, didx1_v,
               out_sp, semi, sem0, sem1):
    c = lax.axis_index("c")
    s = lax.axis_index("s")

    def zr(i, _):
        for j in range(D // L):
            rows0_v[i, pl.ds(j * L, L)] = jnp.zeros((L,), jnp.float32)
        return 0

    def unpack(b, sidx, didx):
        for j in range(K // L):
            v = eidx_v[b, pl.ds(j * L, L)]
            sidx[pl.ds(j * L, L)] = v & 0xFFFF
            didx[pl.ds(j * L, L)] = lax.shift_right_logical(v, 16)

    def fire(b, sidx, didx, rows, sem):
        unpack(b, sidx, didx)
        pltpu.async_copy(g_hbm.at[sidx], rows, sem)

    def wait(rows, sem):
        pltpu.make_async_copy(g_hbm.at[sidx0_v], rows, sem).wait()

    def scat(didx, rows):
        pltpu.sync_copy(rows, out_sp.at[didx], add=True)

    def run(base_blk, nb):
        ci = pltpu.async_copy(ep_hbm.at[pl.ds(base_blk, nb)],
                              eidx_v.at[pl.ds(0, nb)], semi)
        lax.fori_loop(0, K, zr, 0)
        for t in range(RPT // K):
            pltpu.sync_copy(rows0_v, out_sp.at[pl.ds(s * RPT + t * K, K)])
        ci.wait()
        plsc.subcore_barrier()

        fire(0, sidx0_v, didx0_v, rows0_v, sem0)
        nfull = (nb - 1) // 2

        def body(sb, _):
            fire(2 * sb + 1, sidx1_v, didx1_v, rows1_v, sem1)
            wait(rows0_v, sem0)
            scat(didx0_v, rows0_v)
            fire(2 * sb + 2, sidx0_v, didx0_v, rows0_v, sem0)
            wait(rows1_v, sem1)
            scat(didx1_v, rows1_v)
            return 0

        lax.fori_loop(0, nfull, body, 0)
        if nb % 2 == 1:
            wait(rows0_v, sem0)
            scat(didx0_v, rows0_v)
        else:
            fire(nb - 1, sidx1_v, didx1_v, rows1_v, sem1)
            wait(rows0_v, sem0)
            scat(didx0_v, rows0_v)
            wait(rows1_v, sem1)
            scat(didx1_v, rows1_v)

    @pl.when(c == 0)
    def _():
        run(s * B0, B0)

    @pl.when(c == 1)
    def _():
        run(NS * B0 + s * B1, B1)

    plsc.subcore_barrier()
    for t in range(RPT // K):
        r0 = s * RPT + t * K
        pltpu.sync_copy(out_sp.at[pl.ds(r0, K)], rows0_v)
        pltpu.sync_copy(rows0_v, outp_hbm.at[c, pl.ds(r0, K)])


def _dinv_col(degp_ref, nrow):
    i = pl.program_id(0)
    nr = nrow // 128
    d = (degp_ref[0, pl.ds(nr * i, nr), :]
         + degp_ref[1, pl.ds(nr * i, nr), :] + 1.0)
    dinv = lax.rsqrt(d)
    ri = lax.broadcasted_iota(jnp.int32, (nrow, 128), 0)
    li = lax.broadcasted_iota(jnp.int32, (nrow, 128), 1)
    hi = ri // 128
    o = jnp.broadcast_to(dinv[0:1, :], (nrow, 128))
    for k in range(1, nr):
        o = jnp.where(hi == k, jnp.broadcast_to(dinv[k:k + 1, :],
                                                (nrow, 128)), o)
    sel = li == (ri % 128)
    return jnp.sum(jnp.where(sel, o, 0.0), axis=1, keepdims=True)


def _scale_body(degp, x, W, g_out, dinvb_out):
    nrow = x.shape[0]
    dinv = _dinv_col(degp, nrow)
    h = jnp.dot(x[...], W[...], preferred_element_type=jnp.float32)
    g_out[...] = h * dinv
    dinvb_out[...] = jnp.broadcast_to(dinv, (nrow, 128))


def _fuse_body(dinvb, p, g, b, W, g2_out):
    dinv = dinvb[...]
    y = dinv * (p[0] + p[1] + g[...]) + b[...]
    a = jnp.maximum(y, 0.0)
    g2_out[...] = dinv * jnp.dot(a, W[...], preferred_element_type=jnp.float32)


def _finish_body(dinvb, q, g2, b, x, out):
    dinv = dinvb[...]
    y = dinv * (q[0] + q[1] + g2[...]) + b[...]
    out[...] = jnp.maximum(y, 0.0) + x[...]


@jax.jit
def kernel(x, edge_index, W1, b1, W2, b2):
    if x.ndim == 3:
        x = jnp.squeeze(x, axis=1)
    N, D = x.shape
    E = edge_index.shape[1]

    RPAD = _ceil_to(N, NS * K)
    RPT = RPAD // NS
    DR = RPAD // 128
    TB = _ceil_to(-(-E // K), 16 * NS)
    BPC = TB // NS
    B0 = BPC // 2
    B1 = BPC - B0
    BD = TB // NW
    E_pad = TB * K
    pad_row = jnp.int32(RPAD - 1)

    pad_i = jnp.arange(E_pad - E, dtype=jnp.int32)
    pad_spread = N + pad_i % jnp.int32(max(RPAD - N, 1))
    se = jnp.concatenate([edge_index[0], pad_spread])
    de = jnp.concatenate([edge_index[1], pad_spread])
    ep = jnp.bitwise_or(jnp.left_shift(de, 16), se).reshape(TB, K)
    xp = jnp.pad(x, ((0, RPAD - N), (0, 0)))

    mesh = plsc.VectorSubcoreMesh(core_axis_name="c", subcore_axis_name="s")

    deg_call = pl.kernel(
        functools.partial(_deg_body, BD, RPAD),
        out_type=jax.ShapeDtypeStruct((NC, DR, 128), jnp.float32),
        mesh=mesh,
        compiler_params=pltpu.CompilerParams(needs_layout_passes=False),
        scratch_types=[
            pltpu.VMEM((BD, K), jnp.int32),
            pltpu.VMEM((RPAD // 2 * L + L,), jnp.float32),
            pltpu.VMEM((DR, 128), jnp.float32),
            pltpu.VMEM((DR, 128), jnp.float32),
            pltpu.VMEM((DR,), jnp.int32),
            pltpu.VMEM_SHARED((DR, 128), jnp.float32),
            pltpu.SemaphoreType.DMA,
        ],
    )
    degp = deg_call(ep)

    scat_call = pl.kernel(
        functools.partial(_scat_body, B0, B1, RPT, D),
        out_type=jax.ShapeDtypeStruct((NC, RPAD, D), jnp.float32),
        mesh=mesh,
        scratch_types=[
            pltpu.VMEM((max(B0, B1), K), jnp.int32),
            pltpu.VMEM((K, D), jnp.float32),
            pltpu.VMEM((K, D), jnp.float32),
            pltpu.VMEM((K,), jnp.int32),
            pltpu.VMEM((K,), jnp.int32),
            pltpu.VMEM((K,), jnp.int32),
            pltpu.VMEM((K,), jnp.int32),
            pltpu.VMEM_SHARED((RPAD, D), jnp.float32),
            pltpu.SemaphoreType.DMA,
            pltpu.SemaphoreType.DMA,
            pltpu.SemaphoreType.DMA,
        ],
    )

    BR = 512
    grid = (RPAD // BR,)
    degp_spec = pl.BlockSpec((NC, DR, 128), lambda i: (0, 0, 0))
    row_spec = pl.BlockSpec((BR, D), lambda i: (i, 0))
    p_spec = pl.BlockSpec((NC, BR, D), lambda i: (0, i, 0))
    w_spec = pl.BlockSpec((D, D), lambda i: (0, 0))
    b_spec = pl.BlockSpec((1, D), lambda i: (0, 0))
    rows_out = jax.ShapeDtypeStruct((RPAD, D), jnp.float32)

    g1, dinvb = pl.pallas_call(
        _scale_body, grid=grid,
        in_specs=[degp_spec, row_spec, w_spec],
        out_specs=[row_spec, row_spec], out_shape=[rows_out, rows_out],
    )(degp, xp, W1)

    p = scat_call(g1, ep)

    g2 = pl.pallas_call(
        _fuse_body, grid=grid,
        in_specs=[row_spec, p_spec, row_spec, b_spec, w_spec],
        out_specs=row_spec, out_shape=rows_out,
    )(dinvb, p, g1, b1.reshape(1, D), W2)

    q = scat_call(g2, ep)

    BF = next((b for b in (512, 400, 256, 200, 128, 80, 8) if N % b == 0),
              None)
    NF = N if BF else RPAD
    BF = BF or BR
    fr_spec = pl.BlockSpec((BF, D), lambda i: (i, 0))
    fp_spec = pl.BlockSpec((NC, BF, D), lambda i: (0, i, 0))
    fb_spec = pl.BlockSpec((1, D), lambda i: (0, 0))
    out = pl.pallas_call(
        _finish_body, grid=(NF // BF,),
        in_specs=[fr_spec, fp_spec, fr_spec, fb_spec, fr_spec],
        out_specs=fr_spec,
        out_shape=jax.ShapeDtypeStruct((NF, D), jnp.float32),
    )(dinvb, q, g2, b2.reshape(1, D), xp)

    return out[:N]


# speedup vs baseline: 3.1069x; 1.0018x over previous
import functools

import jax
import jax.numpy as jnp
from jax import lax
from jax.experimental import pallas as pl
from jax.experimental.pallas import tpu as pltpu
from jax.experimental.pallas import tpu_sc as plsc

NC = 2
NS = 16
L = 16
NW = NC * NS
K = 128


def _ceil_to(a, m):
    return (a + m - 1) // m * m


def _deg_body(B, RPAD, de_hbm, degp_hbm, didx_v, dl_v, deg_v, db_v, idr_v,
              deg_sp, semi):
    c = lax.axis_index("c")
    s = lax.axis_index("s")
    w = s * NC + c
    R2 = RPAD // 2
    DR = RPAD // 128
    ci = pltpu.async_copy(de_hbm.at[pl.ds(w * B, B)], didx_v, semi)

    def zero_rows(ref, nrow, ncol):
        def zr(i, _):
            for j in range(ncol // L):
                ref[i, pl.ds(j * L, L)] = jnp.zeros((L,), jnp.float32)
            return 0
        lax.fori_loop(0, nrow, zr, 0)

    zero_rows(deg_v, DR, 128)
    for g in range(DR // L):
        idr_v[pl.ds(g * L, L)] = lax.iota(jnp.int32, L) + g * L
    @pl.when(s == 0)
    def _():
        pltpu.sync_copy(deg_v, deg_sp)
    plsc.subcore_barrier()

    lane = lax.iota(jnp.int32, L)
    for p in range(2):
        lo = p * R2

        def zf(i, _):
            for j in range(8):
                dl_v[pl.ds(i * 128 + j * L, L)] = jnp.zeros((L,), jnp.float32)
            return 0

        lax.fori_loop(0, R2 // 8, zf, 0)
        if p == 0:
            ci.wait()

        def grp(i, _):
            v = didx_v[i // (K // L), pl.ds((i % (K // L)) * L, L)]
            idx = lax.shift_right_logical(v, 16)
            m = (idx >= lo) & (idx < lo + R2)
            fi = jnp.where(m, (idx - lo) * L + lane, R2 * L + lane)
            cur = plsc.load_gather(dl_v, [fi])
            plsc.store_scatter(dl_v, [fi], cur + 1.0)
            return 0

        lax.fori_loop(0, B * (K // L), grp, 0)

        def red(rr, _):
            for j in range(8):
                p0 = rr * 128 + j * L
                acc = jnp.zeros((L,), jnp.float32)
                for l in range(L):
                    acc = acc + plsc.load_gather(dl_v, [(p0 + lane) * L + l])
                deg_v[lo // 128 + rr, pl.ds(j * L, L)] = acc
            return 0

        lax.fori_loop(0, R2 // 128, red, 0)

    pltpu.sync_copy(deg_v, deg_sp.at[idr_v], add=True)
    plsc.subcore_barrier()

    @pl.when(s == 0)
    def _():
        pltpu.sync_copy(deg_sp, db_v)
        pltpu.sync_copy(db_v, degp_hbm.at[c])


def _scat_body(B0, B1, RPT, D, g_hbm, ep_hbm, outp_hbm,
               eidx_v, rows0_v, rows1_v, sidx0_v, sidx1_v, didx0_v, didx1_v,
               out_sp, semi, sem0, sem1):
    c = lax.axis_index("c")
    s = lax.axis_index("s")

    def zr(i, _):
        for j in range(D // L):
            rows0_v[i, pl.ds(j * L, L)] = jnp.zeros((L,), jnp.float32)
        return 0

    def unpack(b, sidx, didx):
        for j in range(K // L):
            v = eidx_v[b, pl.ds(j * L, L)]
            sidx[pl.ds(j * L, L)] = v & 0xFFFF
            didx[pl.ds(j * L, L)] = lax.shift_right_logical(v, 16)

    def fire(b, sidx, didx, rows, sem):
        unpack(b, sidx, didx)
        pltpu.async_copy(g_hbm.at[sidx], rows, sem)

    def wait(rows, sem):
        pltpu.make_async_copy(g_hbm.at[sidx0_v], rows, sem).wait()

    def scat(didx, rows):
        pltpu.sync_copy(rows, out_sp.at[didx], add=True)

    def run(base_blk, nb):
        ci = pltpu.async_copy(ep_hbm.at[pl.ds(base_blk, nb)],
                              eidx_v.at[pl.ds(0, nb)], semi)
        lax.fori_loop(0, K, zr, 0)
        for t in range(RPT // K):
            pltpu.sync_copy(rows0_v, out_sp.at[pl.ds(s * RPT + t * K, K)])
        ci.wait()
        plsc.subcore_barrier()

        fire(0, sidx0_v, didx0_v, rows0_v, sem0)
        nfull = (nb - 1) // 2

        def body(sb, _):
            fire(2 * sb + 1, sidx1_v, didx1_v, rows1_v, sem1)
            wait(rows0_v, sem0)
            scat(didx0_v, rows0_v)
            fire(2 * sb + 2, sidx0_v, didx0_v, rows0_v, sem0)
            wait(rows1_v, sem1)
            scat(didx1_v, rows1_v)
            return 0

        lax.fori_loop(0, nfull, body, 0)
        if nb % 2 == 1:
            wait(rows0_v, sem0)
            scat(didx0_v, rows0_v)
        else:
            fire(nb - 1, sidx1_v, didx1_v, rows1_v, sem1)
            wait(rows0_v, sem0)
            scat(didx0_v, rows0_v)
            wait(rows1_v, sem1)
            scat(didx1_v, rows1_v)

    @pl.when(c == 0)
    def _():
        run(s * B0, B0)

    @pl.when(c == 1)
    def _():
        run(NS * B0 + s * B1, B1)

    plsc.subcore_barrier()
    for t in range(RPT // K):
        r0 = s * RPT + t * K
        pltpu.sync_copy(out_sp.at[pl.ds(r0, K)], rows0_v)
        pltpu.sync_copy(rows0_v, outp_hbm.at[c, pl.ds(r0, K)])


def _dinv_col(degp_ref, nrow):
    i = pl.program_id(0)
    nr = nrow // 128
    d = (degp_ref[0, pl.ds(nr * i, nr), :]
         + degp_ref[1, pl.ds(nr * i, nr), :] + 1.0)
    dinv = lax.rsqrt(d)
    ri = lax.broadcasted_iota(jnp.int32, (nrow, 128), 0)
    li = lax.broadcasted_iota(jnp.int32, (nrow, 128), 1)
    hi = ri // 128
    o = jnp.broadcast_to(dinv[0:1, :], (nrow, 128))
    for k in range(1, nr):
        o = jnp.where(hi == k, jnp.broadcast_to(dinv[k:k + 1, :],
                                                (nrow, 128)), o)
    sel = li == (ri % 128)
    return jnp.sum(jnp.where(sel, o, 0.0), axis=1, keepdims=True)


def _mm_body(x, W, h_out):
    h_out[...] = jnp.dot(x[...], W[...], preferred_element_type=jnp.float32)


def _scale_body(degp, h, g_out, dinvb_out):
    nrow = h.shape[0]
    dinv = _dinv_col(degp, nrow)
    g_out[...] = h[...] * dinv
    dinvb_out[...] = jnp.broadcast_to(dinv, (nrow, 128))


def _fuse_body(dinvb, p, g, b, W, g2_out):
    dinv = dinvb[...]
    y = dinv * (p[0] + p[1] + g[...]) + b[...]
    a = jnp.maximum(y, 0.0)
    g2_out[...] = dinv * jnp.dot(a, W[...], preferred_element_type=jnp.float32)


def _finish_body(dinvb, q, g2, b, x, out):
    dinv = dinvb[...]
    y = dinv * (q[0] + q[1] + g2[...]) + b[...]
    out[...] = jnp.maximum(y, 0.0) + x[...]


@jax.jit
def kernel(x, edge_index, W1, b1, W2, b2):
    if x.ndim == 3:
        x = jnp.squeeze(x, axis=1)
    N, D = x.shape
    E = edge_index.shape[1]

    RPAD = _ceil_to(N, NS * K)
    RPT = RPAD // NS
    DR = RPAD // 128
    TB = _ceil_to(-(-E // K), 16 * NS)
    BPC = TB // NS
    B0 = BPC // 2
    B1 = BPC - B0
    BD = TB // NW
    E_pad = TB * K
    pad_row = jnp.int32(RPAD - 1)

    pad_i = jnp.arange(E_pad - E, dtype=jnp.int32)
    pad_spread = N + pad_i % jnp.int32(max(RPAD - N, 1))
    se = jnp.concatenate([edge_index[0], pad_spread])
    de = jnp.concatenate([edge_index[1], pad_spread])
    ep = jnp.bitwise_or(jnp.left_shift(de, 16), se).reshape(TB, K)
    xp = jnp.pad(x, ((0, RPAD - N), (0, 0)))

    mesh = plsc.VectorSubcoreMesh(core_axis_name="c", subcore_axis_name="s")

    deg_call = pl.kernel(
        functools.partial(_deg_body, BD, RPAD),
        out_type=jax.ShapeDtypeStruct((NC, DR, 128), jnp.float32),
        mesh=mesh,
        compiler_params=pltpu.CompilerParams(needs_layout_passes=False),
        scratch_types=[
            pltpu.VMEM((BD, K), jnp.int32),
            pltpu.VMEM((RPAD // 2 * L + L,), jnp.float32),
            pltpu.VMEM((DR, 128), jnp.float32),
            pltpu.VMEM((DR, 128), jnp.float32),
            pltpu.VMEM((DR,), jnp.int32),
            pltpu.VMEM_SHARED((DR, 128), jnp.float32),
            pltpu.SemaphoreType.DMA,
        ],
    )
    degp = deg_call(ep)

    scat_call = pl.kernel(
        functools.partial(_scat_body, B0, B1, RPT, D),
        out_type=jax.ShapeDtypeStruct((NC, RPAD, D), jnp.float32),
        mesh=mesh,
        scratch_types=[
            pltpu.VMEM((max(B0, B1), K), jnp.int32),
            pltpu.VMEM((K, D), jnp.float32),
            pltpu.VMEM((K, D), jnp.float32),
            pltpu.VMEM((K,), jnp.int32),
            pltpu.VMEM((K,), jnp.int32),
            pltpu.VMEM((K,), jnp.int32),
            pltpu.VMEM((K,), jnp.int32),
            pltpu.VMEM_SHARED((RPAD, D), jnp.float32),
            pltpu.SemaphoreType.DMA,
            pltpu.SemaphoreType.DMA,
            pltpu.SemaphoreType.DMA,
        ],
    )

    BR = 512
    grid = (RPAD // BR,)
    degp_spec = pl.BlockSpec((NC, DR, 128), lambda i: (0, 0, 0))
    row_spec = pl.BlockSpec((BR, D), lambda i: (i, 0))
    p_spec = pl.BlockSpec((NC, BR, D), lambda i: (0, i, 0))
    w_spec = pl.BlockSpec((D, D), lambda i: (0, 0))
    b_spec = pl.BlockSpec((1, D), lambda i: (0, 0))
    rows_out = jax.ShapeDtypeStruct((RPAD, D), jnp.float32)

    h1 = pl.pallas_call(
        _mm_body, grid=grid,
        in_specs=[row_spec, w_spec],
        out_specs=row_spec, out_shape=rows_out,
    )(xp, W1)

    g1, dinvb = pl.pallas_call(
        _scale_body, grid=grid,
        in_specs=[degp_spec, row_spec],
        out_specs=[row_spec, row_spec], out_shape=[rows_out, rows_out],
    )(degp, h1)

    p = scat_call(g1, ep)

    g2 = pl.pallas_call(
        _fuse_body, grid=grid,
        in_specs=[row_spec, p_spec, row_spec, b_spec, w_spec],
        out_specs=row_spec, out_shape=rows_out,
    )(dinvb, p, g1, b1.reshape(1, D), W2)

    q = scat_call(g2, ep)

    BF = next((b for b in (512, 400, 256, 200, 128, 80, 8) if N % b == 0),
              None)
    NF = N if BF else RPAD
    BF = BF or BR
    fr_spec = pl.BlockSpec((BF, D), lambda i: (i, 0))
    fp_spec = pl.BlockSpec((NC, BF, D), lambda i: (0, i, 0))
    fb_spec = pl.BlockSpec((1, D), lambda i: (0, 0))
    out = pl.pallas_call(
        _finish_body, grid=(NF // BF,),
        in_specs=[fr_spec, fp_spec, fr_spec, fb_spec, fr_spec],
        out_specs=fr_spec,
        out_shape=jax.ShapeDtypeStruct((NF, D), jnp.float32),
    )(dinvb, q, g2, b2.reshape(1, D), xp)

    return out[:N]
